# baseline (speedup 1.0000x reference)
import functools

import jax
import jax.numpy as jnp
from jax import lax
from jax.experimental import pallas as pl
from jax.experimental.pallas import tpu as pltpu
from jax.experimental.pallas import tpu_sc as plsc

TEMP = 0.07
N = 16384
D = 128
DE = 136
P = 1024
BLK = 2048
NBLK = N // BLK

NC, NS, L = 2, 16, 16
RPT = N // (NC * NS)
CH = 128
NCH = RPT // CH
PPT = P // NS


def _norm_body(feat_ref, fn_ref):
    f = feat_ref[...]
    ss = jnp.sum(f * f, axis=1, keepdims=True)
    fn_ref[...] = f * lax.rsqrt(jnp.maximum(ss, 1e-24))


_sc_mesh = plsc.VectorSubcoreMesh(core_axis_name="c", subcore_axis_name="s")


@functools.partial(
    pl.kernel,
    mesh=_sc_mesh,
    out_type=[
        jax.ShapeDtypeStruct((NC, P, D), jnp.float32),
        jax.ShapeDtypeStruct((NC, P, D), jnp.float32),
    ],
    scratch_types=[
        pltpu.VMEM((CH, D), jnp.float32),
        pltpu.VMEM((CH, D), jnp.float32),
        pltpu.VMEM((CH,), jnp.int32),
        pltpu.VMEM((CH,), jnp.int32),
        pltpu.VMEM((CH, D), jnp.float32),
        pltpu.VMEM((PPT, D), jnp.float32),
        pltpu.VMEM_SHARED((P, D), jnp.float32),
        pltpu.VMEM_SHARED((P, D), jnp.float32),
        pltpu.SemaphoreType.DMA,
        pltpu.SemaphoreType.DMA,
    ],
)
def _sc_segsum(fn_hbm, lab_hbm, ones_hbm, zeros_hbm, pp_out, cc_out,
               rows_v0, rows_v1, idx_v0, idx_v1, ones_v, pstage_v,
               pacc_sh, cacc_sh, gsem, ssem):
    c = lax.axis_index("c")
    s = lax.axis_index("s")
    base = (c * NS + s) * RPT
    rows_b = (rows_v0, rows_v1)
    idx_b = (idx_v0, idx_v1)

    h_ones = pltpu.async_copy(ones_hbm, ones_v, gsem)
    h_z0 = pltpu.async_copy(zeros_hbm, pacc_sh.at[pl.ds(s * PPT, PPT)], gsem)
    h_z1 = pltpu.async_copy(zeros_hbm, cacc_sh.at[pl.ds(s * PPT, PPT)], gsem)
    h_ones.wait()
    h_z0.wait()
    h_z1.wait()
    plsc.subcore_barrier()

    def _start_gather(j):
        hi = pltpu.async_copy(
            lab_hbm.at[pl.ds(base + j * CH, CH)], idx_b[j % 2], gsem)
        hr = pltpu.async_copy(
            fn_hbm.at[pl.ds(base + j * CH, CH)], rows_b[j % 2], gsem)
        return hi, hr

    handles = _start_gather(0)
    scat = []
    for j in range(NCH):
        handles[0].wait()
        handles[1].wait()
        riv, iiv = rows_b[j % 2], idx_b[j % 2]
        if j + 1 < NCH:
            handles = _start_gather(j + 1)
        scat.append(pltpu.async_copy(riv, pacc_sh.at[iiv], ssem, add=True))
        scat.append(pltpu.async_copy(ones_v, cacc_sh.at[iiv], ssem, add=True))

    for h in scat:
        h.wait()
    plsc.subcore_barrier()

    pltpu.sync_copy(pacc_sh.at[pl.ds(s * PPT, PPT)], pstage_v)
    pltpu.sync_copy(pstage_v, pp_out.at[c, pl.ds(s * PPT, PPT)])
    pltpu.sync_copy(cacc_sh.at[pl.ds(s * PPT, PPT)], pstage_v)
    pltpu.sync_copy(pstage_v, cc_out.at[c, pl.ds(s * PPT, PPT)])


def _stage2_body(fn_ref, pp_ref, cc_ref, lab_ref, loss_ref, ps_ref):
    i = pl.program_id(0)

    @pl.when(i == 0)
    def _():
        protos = pp_ref[0] + pp_ref[1]
        cnt = cc_ref[0, :, 0:1] + cc_ref[1, :, 0:1]
        inv = 1.0 / (TEMP * (cnt + 1e-9))
        pscale = protos * inv
        bias = jnp.where(cnt > 0.0, 0.0, -1e30)
        ps_ref[...] = jnp.concatenate(
            [pscale, bias, jnp.zeros((P, DE - D - 1), jnp.float32)], axis=1)

    fn = fn_ref[...]
    fn_ext = jnp.concatenate([fn, jnp.ones((BLK, DE - D), jnp.float32)], axis=1)
    z = lax.dot_general(fn_ext, ps_ref[...], (((1,), (1,)), ((), ())),
                        preferred_element_type=jnp.float32)
    s = jnp.sum(jnp.exp(z), axis=1, keepdims=True)
    lab = lab_ref[0, 0, :]
    oh = (lab[:, None] == lax.broadcasted_iota(jnp.int32, (BLK, P), 1)
          ).astype(jnp.float32)
    tgt = jnp.sum(z * oh, axis=1, keepdims=True)
    part = jnp.sum(jnp.log(s) - tgt, axis=0, keepdims=True)

    @pl.when(i == 0)
    def _():
        loss_ref[...] = part

    @pl.when(i > 0)
    def _():
        loss_ref[...] += part

    @pl.when(i == NBLK - 1)
    def _():
        loss_ref[...] = loss_ref[...] / N


def kernel(features, class_labels, concept_labels):
    del class_labels
    lab3 = concept_labels.reshape(NBLK, 1, BLK)

    fn = pl.pallas_call(
        _norm_body,
        grid=(NBLK,),
        in_specs=[pl.BlockSpec((BLK, D), lambda i: (i, 0))],
        out_specs=pl.BlockSpec((BLK, D), lambda i: (i, 0)),
        out_shape=jax.ShapeDtypeStruct((N, D), jnp.float32),
    )(features)

    ones_blk = jnp.ones((CH, D), jnp.float32)
    zeros_blk = jnp.zeros((PPT, D), jnp.float32)
    pp, cc = _sc_segsum(fn, concept_labels, ones_blk, zeros_blk)

    loss = pl.pallas_call(
        _stage2_body,
        grid=(NBLK,),
        in_specs=[
            pl.BlockSpec((BLK, D), lambda i: (i, 0)),
            pl.BlockSpec((NC, P, D), lambda i: (0, 0, 0)),
            pl.BlockSpec((NC, P, D), lambda i: (0, 0, 0)),
            pl.BlockSpec((1, 1, BLK), lambda i: (i, 0, 0)),
        ],
        out_specs=pl.BlockSpec((1, 1), lambda i: (0, 0)),
        out_shape=jax.ShapeDtypeStruct((1, 1), jnp.float32),
        scratch_shapes=[pltpu.VMEM((P, DE), jnp.float32)],
    )(fn, pp, cc, lab3)

    return loss[0, 0]


# speedup vs baseline: 1.0221x; 1.0221x over previous
import functools

import jax
import jax.numpy as jnp
from jax import lax
from jax.experimental import pallas as pl
from jax.experimental.pallas import tpu as pltpu
from jax.experimental.pallas import tpu_sc as plsc

TEMP = 0.07
N = 16384
D = 128
DE = 136
P = 1024
BLK = 2048
NBLK = N // BLK

NC, NS, L = 2, 16, 16
RPT = N // (NC * NS)
CH = 128
NCH = RPT // CH
PPT = P // NS


def _norm_body(feat_ref, fn_ref, ones_ref, zeros_ref):
    f = feat_ref[...]
    ss = jnp.sum(f * f, axis=1, keepdims=True)
    fn_ref[...] = f * lax.rsqrt(jnp.maximum(ss, 1e-24))

    @pl.when(pl.program_id(0) == 0)
    def _():
        ones_ref[...] = jnp.ones((CH, D), jnp.float32)
        zeros_ref[...] = jnp.zeros((PPT, D), jnp.float32)


_sc_mesh = plsc.VectorSubcoreMesh(core_axis_name="c", subcore_axis_name="s")


@functools.partial(
    pl.kernel,
    mesh=_sc_mesh,
    out_type=[
        jax.ShapeDtypeStruct((NC, P, D), jnp.float32),
        jax.ShapeDtypeStruct((NC, P, D), jnp.float32),
    ],
    scratch_types=[
        pltpu.VMEM((CH, D), jnp.float32),
        pltpu.VMEM((CH, D), jnp.float32),
        pltpu.VMEM((CH,), jnp.int32),
        pltpu.VMEM((CH,), jnp.int32),
        pltpu.VMEM((CH, D), jnp.float32),
        pltpu.VMEM((PPT, D), jnp.float32),
        pltpu.VMEM_SHARED((P, D), jnp.float32),
        pltpu.VMEM_SHARED((P, D), jnp.float32),
        pltpu.SemaphoreType.DMA,
        pltpu.SemaphoreType.DMA,
    ],
)
def _sc_segsum(fn_hbm, lab_hbm, ones_hbm, zeros_hbm, pp_out, cc_out,
               rows_v0, rows_v1, idx_v0, idx_v1, ones_v, pstage_v,
               pacc_sh, cacc_sh, gsem, ssem):
    c = lax.axis_index("c")
    s = lax.axis_index("s")
    base = (c * NS + s) * RPT
    rows_b = (rows_v0, rows_v1)
    idx_b = (idx_v0, idx_v1)

    h_ones = pltpu.async_copy(ones_hbm, ones_v, gsem)
    h_z0 = pltpu.async_copy(zeros_hbm, pacc_sh.at[pl.ds(s * PPT, PPT)], gsem)
    h_z1 = pltpu.async_copy(zeros_hbm, cacc_sh.at[pl.ds(s * PPT, PPT)], gsem)
    h_ones.wait()
    h_z0.wait()
    h_z1.wait()
    plsc.subcore_barrier()

    def _start_gather(j):
        hi = pltpu.async_copy(
            lab_hbm.at[pl.ds(base + j * CH, CH)], idx_b[j % 2], gsem)
        hr = pltpu.async_copy(
            fn_hbm.at[pl.ds(base + j * CH, CH)], rows_b[j % 2], gsem)
        return hi, hr

    handles = _start_gather(0)
    scat = []
    for j in range(NCH):
        handles[0].wait()
        handles[1].wait()
        riv, iiv = rows_b[j % 2], idx_b[j % 2]
        if j + 1 < NCH:
            handles = _start_gather(j + 1)
        scat.append(pltpu.async_copy(riv, pacc_sh.at[iiv], ssem, add=True))
        scat.append(pltpu.async_copy(ones_v, cacc_sh.at[iiv], ssem, add=True))

    for h in scat:
        h.wait()
    plsc.subcore_barrier()

    pltpu.sync_copy(pacc_sh.at[pl.ds(s * PPT, PPT)], pstage_v)
    pltpu.sync_copy(pstage_v, pp_out.at[c, pl.ds(s * PPT, PPT)])
    pltpu.sync_copy(cacc_sh.at[pl.ds(s * PPT, PPT)], pstage_v)
    pltpu.sync_copy(pstage_v, cc_out.at[c, pl.ds(s * PPT, PPT)])


def _stage2_body(fn_ref, pp_ref, cc_ref, lab_ref, loss_ref, ps_ref):
    i = pl.program_id(0)

    @pl.when(i == 0)
    def _():
        protos = pp_ref[0] + pp_ref[1]
        cnt = cc_ref[0, :, 0:1] + cc_ref[1, :, 0:1]
        inv = 1.0 / (TEMP * (cnt + 1e-9))
        pscale = protos * inv
        bias = jnp.where(cnt > 0.0, 0.0, -1e30)
        ps_ref[...] = jnp.concatenate(
            [pscale, bias, jnp.zeros((P, DE - D - 1), jnp.float32)], axis=1)

    fn = fn_ref[...]
    fn_ext = jnp.concatenate([fn, jnp.ones((BLK, DE - D), jnp.float32)], axis=1)
    z = lax.dot_general(fn_ext, ps_ref[...], (((1,), (1,)), ((), ())),
                        preferred_element_type=jnp.float32)
    s = jnp.sum(jnp.exp(z), axis=1, keepdims=True)
    lab = lab_ref[0, 0, :]
    oh = (lab[:, None] == lax.broadcasted_iota(jnp.int32, (BLK, P), 1)
          ).astype(jnp.float32)
    tgt = jnp.sum(z * oh, axis=1, keepdims=True)
    part = jnp.sum(jnp.log(s) - tgt, axis=0, keepdims=True)

    @pl.when(i == 0)
    def _():
        loss_ref[...] = part

    @pl.when(i > 0)
    def _():
        loss_ref[...] += part

    @pl.when(i == NBLK - 1)
    def _():
        loss_ref[...] = loss_ref[...] / N


def kernel(features, class_labels, concept_labels):
    del class_labels
    lab3 = concept_labels.reshape(NBLK, 1, BLK)

    fn, ones_blk, zeros_blk = pl.pallas_call(
        _norm_body,
        grid=(NBLK,),
        in_specs=[pl.BlockSpec((BLK, D), lambda i: (i, 0))],
        out_specs=[
            pl.BlockSpec((BLK, D), lambda i: (i, 0)),
            pl.BlockSpec((CH, D), lambda i: (0, 0)),
            pl.BlockSpec((PPT, D), lambda i: (0, 0)),
        ],
        out_shape=[
            jax.ShapeDtypeStruct((N, D), jnp.float32),
            jax.ShapeDtypeStruct((CH, D), jnp.float32),
            jax.ShapeDtypeStruct((PPT, D), jnp.float32),
        ],
    )(features)

    pp, cc = _sc_segsum(fn, concept_labels, ones_blk, zeros_blk)

    loss = pl.pallas_call(
        _stage2_body,
        grid=(NBLK,),
        in_specs=[
            pl.BlockSpec((BLK, D), lambda i: (i, 0)),
            pl.BlockSpec((NC, P, D), lambda i: (0, 0, 0)),
            pl.BlockSpec((NC, P, D), lambda i: (0, 0, 0)),
            pl.BlockSpec((1, 1, BLK), lambda i: (i, 0, 0)),
        ],
        out_specs=pl.BlockSpec((1, 1), lambda i: (0, 0)),
        out_shape=jax.ShapeDtypeStruct((1, 1), jnp.float32),
        scratch_shapes=[pltpu.VMEM((P, DE), jnp.float32)],
    )(fn, pp, cc, lab3)

    return loss[0, 0]


# speedup vs baseline: 1.1227x; 1.0985x over previous
import functools

import jax
import jax.numpy as jnp
from jax import lax
from jax.experimental import pallas as pl
from jax.experimental.pallas import tpu as pltpu
from jax.experimental.pallas import tpu_sc as plsc

TEMP = 0.07
N = 16384
D = 128
DE = 136
P = 1024
BLK = 2048
NBLK = N // BLK
BLK2 = 4096
NBLK2 = N // BLK2

NC, NS, L = 2, 16, 16
RPT = N // (NC * NS)
CH = 128
NCH = RPT // CH
PPT = P // NS


def _norm_body(feat_ref, fn_ref):
    f = feat_ref[...]
    ss = jnp.sum(f * f, axis=1, keepdims=True)
    fn_ref[...] = f * lax.rsqrt(jnp.maximum(ss, 1e-24))


_sc_mesh = plsc.VectorSubcoreMesh(core_axis_name="c", subcore_axis_name="s")


@functools.partial(
    pl.kernel,
    mesh=_sc_mesh,
    out_type=[
        jax.ShapeDtypeStruct((NC, P, D), jnp.float32),
        jax.ShapeDtypeStruct((NC, P, D), jnp.float32),
    ],
    scratch_types=[
        pltpu.VMEM((CH, D), jnp.float32),
        pltpu.VMEM((CH, D), jnp.float32),
        pltpu.VMEM((CH,), jnp.int32),
        pltpu.VMEM((CH,), jnp.int32),
        pltpu.VMEM((CH, D), jnp.float32),
        pltpu.VMEM((PPT, D), jnp.float32),
        pltpu.VMEM_SHARED((P, D), jnp.float32),
        pltpu.VMEM_SHARED((P, D), jnp.float32),
        pltpu.SemaphoreType.DMA,
        pltpu.SemaphoreType.DMA,
    ],
)
def _sc_segsum(fn_hbm, lab_hbm, pp_out, cc_out,
               rows_v0, rows_v1, idx_v0, idx_v1, ones_v, pstage_v,
               pacc_sh, cacc_sh, gsem, ssem):
    c = lax.axis_index("c")
    s = lax.axis_index("s")
    base = (c * NS + s) * RPT
    rows_b = (rows_v0, rows_v1)
    idx_b = (idx_v0, idx_v1)

    zeros16 = jnp.zeros((L,), jnp.float32)
    ones16 = jnp.ones((L,), jnp.float32)

    def _zero_stage(r, _):
        for k in range(D // L):
            pstage_v[r, pl.ds(k * L, L)] = zeros16
        return 0

    lax.fori_loop(0, PPT, _zero_stage, 0)

    def _ones_row(r, _):
        for k in range(D // L):
            ones_v[r, pl.ds(k * L, L)] = ones16
        return 0

    lax.fori_loop(0, CH, _ones_row, 0)

    pltpu.sync_copy(pstage_v, pacc_sh.at[pl.ds(s * PPT, PPT)])
    pltpu.sync_copy(pstage_v, cacc_sh.at[pl.ds(s * PPT, PPT)])
    plsc.subcore_barrier()

    def _start_gather(j):
        hi = pltpu.async_copy(
            lab_hbm.at[pl.ds(base + j * CH, CH)], idx_b[j % 2], gsem)
        hr = pltpu.async_copy(
            fn_hbm.at[pl.ds(base + j * CH, CH)], rows_b[j % 2], gsem)
        return hi, hr

    handles = _start_gather(0)
    scat = []
    for j in range(NCH):
        handles[0].wait()
        handles[1].wait()
        riv, iiv = rows_b[j % 2], idx_b[j % 2]
        if j + 1 < NCH:
            handles = _start_gather(j + 1)
        scat.append(pltpu.async_copy(riv, pacc_sh.at[iiv], ssem, add=True))
        scat.append(pltpu.async_copy(ones_v, cacc_sh.at[iiv], ssem, add=True))

    for h in scat:
        h.wait()
    plsc.subcore_barrier()

    pltpu.sync_copy(pacc_sh.at[pl.ds(s * PPT, PPT)], pstage_v)
    pltpu.sync_copy(pstage_v, pp_out.at[c, pl.ds(s * PPT, PPT)])
    pltpu.sync_copy(cacc_sh.at[pl.ds(s * PPT, PPT)], pstage_v)
    pltpu.sync_copy(pstage_v, cc_out.at[c, pl.ds(s * PPT, PPT)])


def _stage2_body(fn_ref, pp_ref, cc_ref, lab_ref, loss_ref, ps_ref):
    i = pl.program_id(0)

    @pl.when(i == 0)
    def _():
        protos = pp_ref[0] + pp_ref[1]
        cnt = cc_ref[0, :, 0:1] + cc_ref[1, :, 0:1]
        inv = 1.0 / (TEMP * (cnt + 1e-9))
        pscale = protos * inv
        bias = jnp.where(cnt > 0.0, 0.0, -1e30)
        ps_ref[...] = jnp.concatenate(
            [pscale, bias, jnp.zeros((P, DE - D - 1), jnp.float32)], axis=1)

    fn = fn_ref[...]
    fn_ext = jnp.concatenate([fn, jnp.ones((BLK2, DE - D), jnp.float32)], axis=1)
    z = lax.dot_general(fn_ext, ps_ref[...], (((1,), (1,)), ((), ())),
                        preferred_element_type=jnp.float32)
    s = jnp.sum(jnp.exp(z), axis=1, keepdims=True)
    lab = lab_ref[0, 0, :]
    oh = (lab[:, None] == lax.broadcasted_iota(jnp.int32, (BLK2, P), 1)
          ).astype(jnp.float32)
    tgt = jnp.sum(z * oh, axis=1, keepdims=True)
    part = jnp.sum(jnp.log(s) - tgt, axis=0, keepdims=True)

    @pl.when(i == 0)
    def _():
        loss_ref[...] = part

    @pl.when(i > 0)
    def _():
        loss_ref[...] += part

    @pl.when(i == NBLK2 - 1)
    def _():
        loss_ref[...] = loss_ref[...] / N


def kernel(features, class_labels, concept_labels):
    del class_labels
    lab3 = concept_labels.reshape(NBLK2, 1, BLK2)

    fn = pl.pallas_call(
        _norm_body,
        grid=(NBLK,),
        in_specs=[pl.BlockSpec((BLK, D), lambda i: (i, 0))],
        out_specs=pl.BlockSpec((BLK, D), lambda i: (i, 0)),
        out_shape=jax.ShapeDtypeStruct((N, D), jnp.float32),
    )(features)

    pp, cc = _sc_segsum(fn, concept_labels)

    loss = pl.pallas_call(
        _stage2_body,
        grid=(NBLK2,),
        in_specs=[
            pl.BlockSpec((BLK2, D), lambda i: (i, 0)),
            pl.BlockSpec((NC, P, D), lambda i: (0, 0, 0)),
            pl.BlockSpec((NC, P, D), lambda i: (0, 0, 0)),
            pl.BlockSpec((1, 1, BLK2), lambda i: (i, 0, 0)),
        ],
        out_specs=pl.BlockSpec((1, 1), lambda i: (0, 0)),
        out_shape=jax.ShapeDtypeStruct((1, 1), jnp.float32),
        scratch_shapes=[pltpu.VMEM((P, DE), jnp.float32)],
    )(fn, pp, cc, lab3)

    return loss[0, 0]


# speedup vs baseline: 1.1587x; 1.0320x over previous
import functools

import jax
import jax.numpy as jnp
from jax import lax
from jax.experimental import pallas as pl
from jax.experimental.pallas import tpu as pltpu
from jax.experimental.pallas import tpu_sc as plsc

TEMP = 0.07
N = 16384
D = 128
DE = 136
P = 1024
BLK = 4096
NBLK = N // BLK
BLK2 = 4096
NBLK2 = N // BLK2

NC, NS, L = 2, 16, 16
RPT = N // (NC * NS)
CH = 128
NCH = RPT // CH
PPT = P // NS


def _norm_body(feat_ref, fn_ref):
    f = feat_ref[...]
    ss = jnp.sum(f * f, axis=1, keepdims=True)
    fn_ref[...] = f * lax.rsqrt(jnp.maximum(ss, 1e-24))


_sc_mesh = plsc.VectorSubcoreMesh(core_axis_name="c", subcore_axis_name="s")


@functools.partial(
    pl.kernel,
    mesh=_sc_mesh,
    out_type=[
        jax.ShapeDtypeStruct((NC, P, D), jnp.float32),
        jax.ShapeDtypeStruct((NC, P, D), jnp.float32),
    ],
    scratch_types=[
        pltpu.VMEM((CH, D), jnp.float32),
        pltpu.VMEM((CH, D), jnp.float32),
        pltpu.VMEM((CH,), jnp.int32),
        pltpu.VMEM((CH,), jnp.int32),
        pltpu.VMEM((CH, D), jnp.float32),
        pltpu.VMEM((PPT, D), jnp.float32),
        pltpu.VMEM_SHARED((P, D), jnp.float32),
        pltpu.VMEM_SHARED((P, D), jnp.float32),
        pltpu.SemaphoreType.DMA,
        pltpu.SemaphoreType.DMA,
    ],
)
def _sc_segsum(fn_hbm, lab_hbm, pp_out, cc_out,
               rows_v0, rows_v1, idx_v0, idx_v1, ones_v, pstage_v,
               pacc_sh, cacc_sh, gsem, ssem):
    c = lax.axis_index("c")
    s = lax.axis_index("s")
    base = (c * NS + s) * RPT
    rows_b = (rows_v0, rows_v1)
    idx_b = (idx_v0, idx_v1)

    zeros16 = jnp.zeros((L,), jnp.float32)
    ones16 = jnp.ones((L,), jnp.float32)

    def _zero_stage(r, _):
        for k in range(D // L):
            pstage_v[r, pl.ds(k * L, L)] = zeros16
        return 0

    lax.fori_loop(0, PPT, _zero_stage, 0)

    def _ones_row(r, _):
        for k in range(D // L):
            ones_v[r, pl.ds(k * L, L)] = ones16
        return 0

    lax.fori_loop(0, CH, _ones_row, 0)

    pltpu.sync_copy(pstage_v, pacc_sh.at[pl.ds(s * PPT, PPT)])
    pltpu.sync_copy(pstage_v, cacc_sh.at[pl.ds(s * PPT, PPT)])
    plsc.subcore_barrier()

    def _start_gather(j):
        hi = pltpu.async_copy(
            lab_hbm.at[pl.ds(base + j * CH, CH)], idx_b[j % 2], gsem)
        hr = pltpu.async_copy(
            fn_hbm.at[pl.ds(base + j * CH, CH)], rows_b[j % 2], gsem)
        return hi, hr

    handles = _start_gather(0)
    scat = []
    for j in range(NCH):
        handles[0].wait()
        handles[1].wait()
        riv, iiv = rows_b[j % 2], idx_b[j % 2]
        if j + 1 < NCH:
            handles = _start_gather(j + 1)
        scat.append(pltpu.async_copy(riv, pacc_sh.at[iiv], ssem, add=True))
        scat.append(pltpu.async_copy(ones_v, cacc_sh.at[iiv], ssem, add=True))

    for h in scat:
        h.wait()
    plsc.subcore_barrier()

    pltpu.sync_copy(pacc_sh.at[pl.ds(s * PPT, PPT)], pstage_v)
    pltpu.sync_copy(cacc_sh.at[pl.ds(s * PPT, PPT)], rows_v0.at[pl.ds(0, PPT)])
    h0 = pltpu.async_copy(pstage_v, pp_out.at[c, pl.ds(s * PPT, PPT)], gsem)
    h1 = pltpu.async_copy(rows_v0.at[pl.ds(0, PPT)],
                          cc_out.at[c, pl.ds(s * PPT, PPT)], gsem)
    h0.wait()
    h1.wait()


def _stage2_body(fn_ref, pp_ref, cc_ref, lab_ref, loss_ref, ps_ref):
    i = pl.program_id(0)

    @pl.when(i == 0)
    def _():
        protos = pp_ref[0] + pp_ref[1]
        cnt = cc_ref[0, :, 0:1] + cc_ref[1, :, 0:1]
        inv = 1.0 / (TEMP * (cnt + 1e-9))
        pscale = protos * inv
        bias = jnp.where(cnt > 0.0, 0.0, -1e30)
        ps_ref[...] = jnp.concatenate(
            [pscale, bias, jnp.zeros((P, DE - D - 1), jnp.float32)], axis=1)

    fn = fn_ref[...]
    fn_ext = jnp.concatenate([fn, jnp.ones((BLK2, DE - D), jnp.float32)], axis=1)
    z = lax.dot_general(fn_ext, ps_ref[...], (((1,), (1,)), ((), ())),
                        preferred_element_type=jnp.float32)
    s = jnp.sum(jnp.exp(z), axis=1, keepdims=True)
    lab = lab_ref[0, 0, :]
    oh = (lab[:, None] == lax.broadcasted_iota(jnp.int32, (BLK2, P), 1)
          ).astype(jnp.float32)
    tgt = jnp.sum(z * oh, axis=1, keepdims=True)
    part = jnp.sum(jnp.log(s) - tgt, axis=0, keepdims=True)

    @pl.when(i == 0)
    def _():
        loss_ref[...] = part

    @pl.when(i > 0)
    def _():
        loss_ref[...] += part

    @pl.when(i == NBLK2 - 1)
    def _():
        loss_ref[...] = loss_ref[...] / N


def kernel(features, class_labels, concept_labels):
    del class_labels
    lab3 = concept_labels.reshape(NBLK2, 1, BLK2)

    fn = pl.pallas_call(
        _norm_body,
        grid=(NBLK,),
        in_specs=[pl.BlockSpec((BLK, D), lambda i: (i, 0))],
        out_specs=pl.BlockSpec((BLK, D), lambda i: (i, 0)),
        out_shape=jax.ShapeDtypeStruct((N, D), jnp.float32),
    )(features)

    pp, cc = _sc_segsum(fn, concept_labels)

    loss = pl.pallas_call(
        _stage2_body,
        grid=(NBLK2,),
        in_specs=[
            pl.BlockSpec((BLK2, D), lambda i: (i, 0)),
            pl.BlockSpec((NC, P, D), lambda i: (0, 0, 0)),
            pl.BlockSpec((NC, P, D), lambda i: (0, 0, 0)),
            pl.BlockSpec((1, 1, BLK2), lambda i: (i, 0, 0)),
        ],
        out_specs=pl.BlockSpec((1, 1), lambda i: (0, 0)),
        out_shape=jax.ShapeDtypeStruct((1, 1), jnp.float32),
        scratch_shapes=[pltpu.VMEM((P, DE), jnp.float32)],
    )(fn, pp, cc, lab3)

    return loss[0, 0]
